# baseline (speedup 1.0000x reference)
import functools

import jax
import jax.numpy as jnp
from jax import lax
from jax.experimental import pallas as pl
from jax.experimental.pallas import tpu as pltpu
from jax.experimental.pallas import tpu_sc as plsc

D = 256
HALF = 128
NB = 8
BATCH = 4096
SEQ = 64
NROWS = BATCH * SEQ
TAB = NB * NB * NB
HTAB = NB * NB

_KSC = 1536
_G = 64

_info = plsc.get_sparse_core_info()
_NC, _NS = _info.num_cores, _info.num_subcores
_NW = _NC * _NS
_BPW = _KSC // _NW
_CHUNK_B = 2
_CHUNK_R = _CHUNK_B * SEQ
_NCHUNK = _BPW // _CHUNK_B
_XPW = _BPW * 3 * SEQ


def _tables_body(te_ref, pe_ref, c_ref, tdhi_ref, tdlo_ref):
    te8 = te_ref[...]
    pe = pe_ref[...]
    per = jnp.broadcast_to(pe[None, :, :], (NB, NB, HALF)).reshape(HTAB, HALF)
    ta = jnp.broadcast_to(te8[:, None, :HALF] * 16.0, (NB, NB, HALF)).reshape(HTAB, HALF)
    tb = jnp.broadcast_to(te8[:, None, HALF:] * 16.0, (NB, NB, HALF)).reshape(HTAB, HALF)
    a = ta + per
    b = tb + per
    ca = jnp.broadcast_to(a[:, None, :], (HTAB, NB, HALF)).reshape(TAB, HALF)
    cb = jnp.broadcast_to(b.reshape(NB, NB, HALF)[:, None, :, :],
                          (NB, NB, NB, HALF)).reshape(TAB, HALF)
    c_ref[...] = jnp.concatenate([ca, cb], axis=-1)
    z = jnp.zeros((HTAB, HALF), jnp.float32)
    td = jnp.concatenate(
        [jnp.concatenate([a, z], axis=1), jnp.concatenate([z, b], axis=1)],
        axis=0)
    hi = td.astype(jnp.bfloat16)
    tdhi_ref[...] = hi
    tdlo_ref[...] = (td - hi.astype(jnp.float32)).astype(jnp.bfloat16)


_mesh = plsc.VectorSubcoreMesh(core_axis_name="c", subcore_axis_name="s")


@functools.partial(
    pl.kernel,
    mesh=_mesh,
    out_type=jax.ShapeDtypeStruct((_KSC * SEQ, D), jnp.float32),
    scratch_types=[
        pltpu.VMEM((_XPW,), jnp.int32),
        pltpu.VMEM((2, _CHUNK_R), jnp.int32),
        pltpu.VMEM((_CHUNK_R, D), jnp.float32),
        pltpu.VMEM((_CHUNK_R, D), jnp.float32),
        pltpu.SemaphoreType.DMA,
        pltpu.SemaphoreType.DMA,
        pltpu.SemaphoreType.DMA,
    ],
)
def _sc_gather(xf_hbm, tab_hbm, out_hbm, xv, idxv, rowsA, rowsB,
               gsem, wsA, wsB):
    wid = lax.axis_index("s") * _NC + lax.axis_index("c")
    pltpu.sync_copy(xf_hbm.at[pl.ds(wid * _XPW, _XPW)], xv)
    rbase = wid * (_BPW * SEQ)

    rows = (rowsA, rowsB)
    wsem = (wsA, wsB)

    def do_chunk(c, p):
        @pl.when(c >= 2)
        def _():
            pltpu.make_async_copy(
                rows[p], out_hbm.at[pl.ds(rbase, _CHUNK_R)], wsem[p]).wait()

        for j in range(_CHUNK_B):
            boff = (c * _CHUNK_B + j) * (3 * SEQ)
            for k in range(SEQ // 16):
                t = xv[pl.ds(boff + k * 16, 16)]
                p1 = xv[pl.ds(boff + SEQ + k * 16, 16)]
                p2 = xv[pl.ds(boff + 2 * SEQ + k * 16, 16)]
                idxv[p, pl.ds(j * SEQ + k * 16, 16)] = t * (NB * NB) + p1 * NB + p2
        pltpu.async_copy(tab_hbm.at[idxv.at[p]], rows[p], gsem).wait()
        pltpu.async_copy(
            rows[p],
            out_hbm.at[pl.ds(rbase + c * _CHUNK_R, _CHUNK_R)],
            wsem[p])

    def body(i, carry):
        do_chunk(2 * i, 0)
        do_chunk(2 * i + 1, 1)
        return carry

    lax.fori_loop(0, _NCHUNK // 2, body, 0)
    for p in range(2):
        pltpu.make_async_copy(
            rows[p], out_hbm.at[pl.ds(rbase, _CHUNK_R)], wsem[p]).wait()


def _tc_body(t_ref, p1_ref, p2_ref, tdhi_ref, tdlo_ref, out_ref):
    ia = t_ref[...] * NB + p1_ref[...]
    ib = t_ref[...] * NB + p2_ref[...]
    lanes = lax.broadcasted_iota(jnp.int32, (_G, SEQ, 2 * HTAB), 2)
    target = jnp.where(lanes < HTAB, ia[:, :, None], ib[:, :, None] + HTAB)
    twohot = (lanes == target).astype(jnp.bfloat16).reshape(_G * SEQ, 2 * HTAB)
    res = (jnp.dot(twohot, tdhi_ref[...], preferred_element_type=jnp.float32)
           + jnp.dot(twohot, tdlo_ref[...], preferred_element_type=jnp.float32))
    out_ref[...] = res.reshape(_G, SEQ, D)


def kernel(x, token_embedding, pe):
    ctab, tdhi, tdlo = pl.pallas_call(
        _tables_body,
        out_shape=(jax.ShapeDtypeStruct((TAB, D), jnp.float32),
                   jax.ShapeDtypeStruct((2 * HTAB, D), jnp.bfloat16),
                   jax.ShapeDtypeStruct((2 * HTAB, D), jnp.bfloat16)),
    )(token_embedding[:NB], pe)

    sc_out = _sc_gather(x[:_KSC].reshape(-1), ctab)

    ntc = (BATCH - _KSC) // _G
    out = pl.pallas_call(
        _tc_body,
        grid=(ntc,),
        in_specs=[
            pl.BlockSpec((_G, SEQ), lambda i: (i, 0)),
            pl.BlockSpec((_G, SEQ), lambda i: (i, 0)),
            pl.BlockSpec((_G, SEQ), lambda i: (i, 0)),
            pl.BlockSpec((2 * HTAB, D), lambda i: (0, 0)),
            pl.BlockSpec((2 * HTAB, D), lambda i: (0, 0)),
        ],
        out_specs=pl.BlockSpec((_G, SEQ, D), lambda i: (_KSC // _G + i, 0, 0)),
        out_shape=jax.ShapeDtypeStruct((BATCH, SEQ, D), jnp.float32),
    )(x[_KSC:, 0, :], x[_KSC:, 1, :], x[_KSC:, 2, :], tdhi, tdlo)
    return lax.dynamic_update_slice(
        out, sc_out.reshape(_KSC, SEQ, D), (0, 0, 0))


# speedup vs baseline: 1.1282x; 1.1282x over previous
import functools

import jax
import jax.numpy as jnp
from jax import lax
from jax.experimental import pallas as pl
from jax.experimental.pallas import tpu as pltpu
from jax.experimental.pallas import tpu_sc as plsc

D = 256
HALF = 128
NB = 8
BATCH = 4096
SEQ = 64
NROWS = BATCH * SEQ
TAB = NB * NB * NB
HTAB = NB * NB

_KSC = 1024
_G = 32

_info = plsc.get_sparse_core_info()
_NC, _NS = _info.num_cores, _info.num_subcores
_NW = _NC * _NS
_BPW = _KSC // _NW
_CHUNK_B = 2
_CHUNK_R = _CHUNK_B * SEQ
_NCHUNK = _BPW // _CHUNK_B
_XPW = _BPW * 3 * SEQ


def _tables_body(te_ref, pe_ref, c_ref, tdhi_ref, tdlo_ref):
    te8 = te_ref[...]
    pe = pe_ref[...]
    per = jnp.broadcast_to(pe[None, :, :], (NB, NB, HALF)).reshape(HTAB, HALF)
    ta = jnp.broadcast_to(te8[:, None, :HALF] * 16.0, (NB, NB, HALF)).reshape(HTAB, HALF)
    tb = jnp.broadcast_to(te8[:, None, HALF:] * 16.0, (NB, NB, HALF)).reshape(HTAB, HALF)
    a = ta + per
    b = tb + per
    ca = jnp.broadcast_to(a[:, None, :], (HTAB, NB, HALF)).reshape(TAB, HALF)
    cb = jnp.broadcast_to(b.reshape(NB, NB, HALF)[:, None, :, :],
                          (NB, NB, NB, HALF)).reshape(TAB, HALF)
    c_ref[...] = jnp.concatenate([ca, cb], axis=-1)
    z = jnp.zeros((HTAB, HALF), jnp.float32)
    td = jnp.concatenate(
        [jnp.concatenate([a, z], axis=1), jnp.concatenate([z, b], axis=1)],
        axis=0)
    hi = td.astype(jnp.bfloat16)
    tdhi_ref[...] = hi
    tdlo_ref[...] = (td - hi.astype(jnp.float32)).astype(jnp.bfloat16)


_mesh = plsc.VectorSubcoreMesh(core_axis_name="c", subcore_axis_name="s")


@functools.partial(
    pl.kernel,
    mesh=_mesh,
    out_type=jax.ShapeDtypeStruct((_KSC * SEQ, D), jnp.float32),
    scratch_types=[
        pltpu.VMEM((_XPW,), jnp.int32),
        pltpu.VMEM((2, _CHUNK_R), jnp.int32),
        pltpu.VMEM((_CHUNK_R, D), jnp.float32),
        pltpu.VMEM((_CHUNK_R, D), jnp.float32),
        pltpu.SemaphoreType.DMA,
        pltpu.SemaphoreType.DMA,
        pltpu.SemaphoreType.DMA,
    ],
)
def _sc_gather(xf_hbm, tab_hbm, out_hbm, xv, idxv, rowsA, rowsB,
               gsem, wsA, wsB):
    wid = lax.axis_index("s") * _NC + lax.axis_index("c")
    pltpu.sync_copy(xf_hbm.at[pl.ds(wid * _XPW, _XPW)], xv)
    rbase = wid * (_BPW * SEQ)

    rows = (rowsA, rowsB)
    wsem = (wsA, wsB)

    def do_chunk(c, p):
        @pl.when(c >= 2)
        def _():
            pltpu.make_async_copy(
                rows[p], out_hbm.at[pl.ds(rbase, _CHUNK_R)], wsem[p]).wait()

        for j in range(_CHUNK_B):
            boff = (c * _CHUNK_B + j) * (3 * SEQ)
            for k in range(SEQ // 16):
                t = xv[pl.ds(boff + k * 16, 16)]
                p1 = xv[pl.ds(boff + SEQ + k * 16, 16)]
                p2 = xv[pl.ds(boff + 2 * SEQ + k * 16, 16)]
                idxv[p, pl.ds(j * SEQ + k * 16, 16)] = t * (NB * NB) + p1 * NB + p2
        pltpu.async_copy(tab_hbm.at[idxv.at[p]], rows[p], gsem).wait()
        pltpu.async_copy(
            rows[p],
            out_hbm.at[pl.ds(rbase + c * _CHUNK_R, _CHUNK_R)],
            wsem[p])

    def body(i, carry):
        do_chunk(2 * i, 0)
        do_chunk(2 * i + 1, 1)
        return carry

    lax.fori_loop(0, _NCHUNK // 2, body, 0)
    for p in range(2):
        pltpu.make_async_copy(
            rows[p], out_hbm.at[pl.ds(rbase, _CHUNK_R)], wsem[p]).wait()


def _tc_body(t_ref, p1_ref, p2_ref, tdhi_ref, tdlo_ref, out_ref):
    ia = t_ref[...] * NB + p1_ref[...]
    ib = t_ref[...] * NB + p2_ref[...]
    lanes = lax.broadcasted_iota(jnp.int32, (_G, SEQ, 2 * HTAB), 2)
    target = jnp.where(lanes < HTAB, ia[:, :, None], ib[:, :, None] + HTAB)
    twohot = (lanes == target).astype(jnp.bfloat16).reshape(_G * SEQ, 2 * HTAB)
    res = (jnp.dot(twohot, tdhi_ref[...], preferred_element_type=jnp.float32)
           + jnp.dot(twohot, tdlo_ref[...], preferred_element_type=jnp.float32))
    out_ref[...] = res.reshape(_G, SEQ, D)


def kernel(x, token_embedding, pe):
    ctab, tdhi, tdlo = pl.pallas_call(
        _tables_body,
        out_shape=(jax.ShapeDtypeStruct((TAB, D), jnp.float32),
                   jax.ShapeDtypeStruct((2 * HTAB, D), jnp.bfloat16),
                   jax.ShapeDtypeStruct((2 * HTAB, D), jnp.bfloat16)),
    )(token_embedding[:NB], pe)

    sc_out = _sc_gather(x[:_KSC].reshape(-1), ctab)

    ntc = (BATCH - _KSC) // _G
    out = pl.pallas_call(
        _tc_body,
        grid=(ntc,),
        in_specs=[
            pl.BlockSpec((_G, SEQ), lambda i: (i, 0)),
            pl.BlockSpec((_G, SEQ), lambda i: (i, 0)),
            pl.BlockSpec((_G, SEQ), lambda i: (i, 0)),
            pl.BlockSpec((2 * HTAB, D), lambda i: (0, 0)),
            pl.BlockSpec((2 * HTAB, D), lambda i: (0, 0)),
        ],
        out_specs=pl.BlockSpec((_G, SEQ, D), lambda i: (_KSC // _G + i, 0, 0)),
        out_shape=jax.ShapeDtypeStruct((BATCH, SEQ, D), jnp.float32),
    )(x[_KSC:, 0, :], x[_KSC:, 1, :], x[_KSC:, 2, :], tdhi, tdlo)
    return lax.dynamic_update_slice(
        out, sc_out.reshape(_KSC, SEQ, D), (0, 0, 0))


# speedup vs baseline: 1.1789x; 1.0450x over previous
import functools

import jax
import jax.numpy as jnp
from jax import lax
from jax.experimental import pallas as pl
from jax.experimental.pallas import tpu as pltpu
from jax.experimental.pallas import tpu_sc as plsc

D = 256
HALF = 128
NB = 8
BATCH = 4096
SEQ = 64
NROWS = BATCH * SEQ
TAB = NB * NB * NB
HTAB = NB * NB

_KSC = 768
_G = 32

_info = plsc.get_sparse_core_info()
_NC, _NS = _info.num_cores, _info.num_subcores
_NW = _NC * _NS
_BPW = _KSC // _NW
_CHUNK_B = 2
_CHUNK_R = _CHUNK_B * SEQ
_NCHUNK = _BPW // _CHUNK_B
_XPW = _BPW * 3 * SEQ


def _tables_body(te_ref, pe_ref, c_ref, tdhi_ref, tdlo_ref):
    te8 = te_ref[...]
    pe = pe_ref[...]
    per = jnp.broadcast_to(pe[None, :, :], (NB, NB, HALF)).reshape(HTAB, HALF)
    ta = jnp.broadcast_to(te8[:, None, :HALF] * 16.0, (NB, NB, HALF)).reshape(HTAB, HALF)
    tb = jnp.broadcast_to(te8[:, None, HALF:] * 16.0, (NB, NB, HALF)).reshape(HTAB, HALF)
    a = ta + per
    b = tb + per
    ca = jnp.broadcast_to(a[:, None, :], (HTAB, NB, HALF)).reshape(TAB, HALF)
    cb = jnp.broadcast_to(b.reshape(NB, NB, HALF)[:, None, :, :],
                          (NB, NB, NB, HALF)).reshape(TAB, HALF)
    c_ref[...] = jnp.concatenate([ca, cb], axis=-1)
    z = jnp.zeros((HTAB, HALF), jnp.float32)
    td = jnp.concatenate(
        [jnp.concatenate([a, z], axis=1), jnp.concatenate([z, b], axis=1)],
        axis=0)
    hi = td.astype(jnp.bfloat16)
    tdhi_ref[...] = hi
    tdlo_ref[...] = (td - hi.astype(jnp.float32)).astype(jnp.bfloat16)


_mesh = plsc.VectorSubcoreMesh(core_axis_name="c", subcore_axis_name="s")


@functools.partial(
    pl.kernel,
    mesh=_mesh,
    out_type=jax.ShapeDtypeStruct((_KSC * SEQ, D), jnp.float32),
    scratch_types=[
        pltpu.VMEM((_XPW,), jnp.int32),
        pltpu.VMEM((2, _CHUNK_R), jnp.int32),
        pltpu.VMEM((_CHUNK_R, D), jnp.float32),
        pltpu.VMEM((_CHUNK_R, D), jnp.float32),
        pltpu.SemaphoreType.DMA,
        pltpu.SemaphoreType.DMA,
        pltpu.SemaphoreType.DMA,
    ],
)
def _sc_gather(xf_hbm, tab_hbm, out_hbm, xv, idxv, rowsA, rowsB,
               gsem, wsA, wsB):
    wid = lax.axis_index("s") * _NC + lax.axis_index("c")
    pltpu.sync_copy(xf_hbm.at[pl.ds(wid * _XPW, _XPW)], xv)
    rbase = wid * (_BPW * SEQ)

    rows = (rowsA, rowsB)
    wsem = (wsA, wsB)

    def do_chunk(c, p):
        @pl.when(c >= 2)
        def _():
            pltpu.make_async_copy(
                rows[p], out_hbm.at[pl.ds(rbase, _CHUNK_R)], wsem[p]).wait()

        for j in range(_CHUNK_B):
            boff = (c * _CHUNK_B + j) * (3 * SEQ)
            for k in range(SEQ // 16):
                t = xv[pl.ds(boff + k * 16, 16)]
                p1 = xv[pl.ds(boff + SEQ + k * 16, 16)]
                p2 = xv[pl.ds(boff + 2 * SEQ + k * 16, 16)]
                idxv[p, pl.ds(j * SEQ + k * 16, 16)] = t * (NB * NB) + p1 * NB + p2
        pltpu.async_copy(tab_hbm.at[idxv.at[p]], rows[p], gsem).wait()
        pltpu.async_copy(
            rows[p],
            out_hbm.at[pl.ds(rbase + c * _CHUNK_R, _CHUNK_R)],
            wsem[p])

    def body(i, carry):
        do_chunk(2 * i, 0)
        do_chunk(2 * i + 1, 1)
        return carry

    lax.fori_loop(0, _NCHUNK // 2, body, 0)
    for p in range(2):
        pltpu.make_async_copy(
            rows[p], out_hbm.at[pl.ds(rbase, _CHUNK_R)], wsem[p]).wait()


def _tc_body(t_ref, p1_ref, p2_ref, tdhi_ref, tdlo_ref, out_ref):
    ia = t_ref[...] * NB + p1_ref[...]
    ib = t_ref[...] * NB + p2_ref[...]
    lanes = lax.broadcasted_iota(jnp.int32, (_G, SEQ, 2 * HTAB), 2)
    target = jnp.where(lanes < HTAB, ia[:, :, None], ib[:, :, None] + HTAB)
    twohot = (lanes == target).astype(jnp.bfloat16).reshape(_G * SEQ, 2 * HTAB)
    res = (jnp.dot(twohot, tdhi_ref[...], preferred_element_type=jnp.float32)
           + jnp.dot(twohot, tdlo_ref[...], preferred_element_type=jnp.float32))
    out_ref[...] = res.reshape(_G, SEQ, D)


def kernel(x, token_embedding, pe):
    ctab, tdhi, tdlo = pl.pallas_call(
        _tables_body,
        out_shape=(jax.ShapeDtypeStruct((TAB, D), jnp.float32),
                   jax.ShapeDtypeStruct((2 * HTAB, D), jnp.bfloat16),
                   jax.ShapeDtypeStruct((2 * HTAB, D), jnp.bfloat16)),
    )(token_embedding[:NB], pe)

    sc_out = _sc_gather(x[:_KSC].reshape(-1), ctab)

    ntc = (BATCH - _KSC) // _G
    out = pl.pallas_call(
        _tc_body,
        grid=(ntc,),
        in_specs=[
            pl.BlockSpec((_G, SEQ), lambda i: (i, 0)),
            pl.BlockSpec((_G, SEQ), lambda i: (i, 0)),
            pl.BlockSpec((_G, SEQ), lambda i: (i, 0)),
            pl.BlockSpec((2 * HTAB, D), lambda i: (0, 0)),
            pl.BlockSpec((2 * HTAB, D), lambda i: (0, 0)),
        ],
        out_specs=pl.BlockSpec((_G, SEQ, D), lambda i: (_KSC // _G + i, 0, 0)),
        out_shape=jax.ShapeDtypeStruct((BATCH, SEQ, D), jnp.float32),
    )(x[_KSC:, 0, :], x[_KSC:, 1, :], x[_KSC:, 2, :], tdhi, tdlo)
    return lax.dynamic_update_slice(
        out, sc_out.reshape(_KSC, SEQ, D), (0, 0, 0))


# speedup vs baseline: 1.2612x; 1.0698x over previous
import functools

import jax
import jax.numpy as jnp
from jax import lax
from jax.experimental import pallas as pl
from jax.experimental.pallas import tpu as pltpu
from jax.experimental.pallas import tpu_sc as plsc

D = 256
HALF = 128
NB = 8
BATCH = 4096
SEQ = 64
NROWS = BATCH * SEQ
TAB = NB * NB * NB
HTAB = NB * NB

_KSC = 512
_G = 32

_info = plsc.get_sparse_core_info()
_NC, _NS = _info.num_cores, _info.num_subcores
_NW = _NC * _NS
_BPW = _KSC // _NW
_CHUNK_B = 2
_CHUNK_R = _CHUNK_B * SEQ
_NCHUNK = _BPW // _CHUNK_B
_XPW = _BPW * 3 * SEQ


def _tables_body(te_ref, pe_ref, c_ref, tdhi_ref, tdlo_ref):
    te8 = te_ref[...]
    pe = pe_ref[...]
    per = jnp.broadcast_to(pe[None, :, :], (NB, NB, HALF)).reshape(HTAB, HALF)
    ta = jnp.broadcast_to(te8[:, None, :HALF] * 16.0, (NB, NB, HALF)).reshape(HTAB, HALF)
    tb = jnp.broadcast_to(te8[:, None, HALF:] * 16.0, (NB, NB, HALF)).reshape(HTAB, HALF)
    a = ta + per
    b = tb + per
    ca = jnp.broadcast_to(a[:, None, :], (HTAB, NB, HALF)).reshape(TAB, HALF)
    cb = jnp.broadcast_to(b.reshape(NB, NB, HALF)[:, None, :, :],
                          (NB, NB, NB, HALF)).reshape(TAB, HALF)
    c_ref[...] = jnp.concatenate([ca, cb], axis=-1)
    z = jnp.zeros((HTAB, HALF), jnp.float32)
    td = jnp.concatenate(
        [jnp.concatenate([a, z], axis=1), jnp.concatenate([z, b], axis=1)],
        axis=0)
    hi = td.astype(jnp.bfloat16)
    tdhi_ref[...] = hi
    tdlo_ref[...] = (td - hi.astype(jnp.float32)).astype(jnp.bfloat16)


_mesh = plsc.VectorSubcoreMesh(core_axis_name="c", subcore_axis_name="s")


@functools.partial(
    pl.kernel,
    mesh=_mesh,
    out_type=jax.ShapeDtypeStruct((_KSC * SEQ, D), jnp.float32),
    scratch_types=[
        pltpu.VMEM((_XPW,), jnp.int32),
        pltpu.VMEM((2, _CHUNK_R), jnp.int32),
        pltpu.VMEM((_CHUNK_R, D), jnp.float32),
        pltpu.VMEM((_CHUNK_R, D), jnp.float32),
        pltpu.SemaphoreType.DMA,
        pltpu.SemaphoreType.DMA,
        pltpu.SemaphoreType.DMA,
    ],
)
def _sc_gather(xf_hbm, tab_hbm, out_hbm, xv, idxv, rowsA, rowsB,
               gsem, wsA, wsB):
    wid = lax.axis_index("s") * _NC + lax.axis_index("c")
    pltpu.sync_copy(xf_hbm.at[pl.ds(wid * _XPW, _XPW)], xv)
    rbase = wid * (_BPW * SEQ)

    rows = (rowsA, rowsB)
    wsem = (wsA, wsB)

    def do_chunk(c, p):
        @pl.when(c >= 2)
        def _():
            pltpu.make_async_copy(
                rows[p], out_hbm.at[pl.ds(rbase, _CHUNK_R)], wsem[p]).wait()

        for j in range(_CHUNK_B):
            boff = (c * _CHUNK_B + j) * (3 * SEQ)
            for k in range(SEQ // 16):
                t = xv[pl.ds(boff + k * 16, 16)]
                p1 = xv[pl.ds(boff + SEQ + k * 16, 16)]
                p2 = xv[pl.ds(boff + 2 * SEQ + k * 16, 16)]
                idxv[p, pl.ds(j * SEQ + k * 16, 16)] = t * (NB * NB) + p1 * NB + p2
        pltpu.async_copy(tab_hbm.at[idxv.at[p]], rows[p], gsem).wait()
        pltpu.async_copy(
            rows[p],
            out_hbm.at[pl.ds(rbase + c * _CHUNK_R, _CHUNK_R)],
            wsem[p])

    def body(i, carry):
        do_chunk(2 * i, 0)
        do_chunk(2 * i + 1, 1)
        return carry

    lax.fori_loop(0, _NCHUNK // 2, body, 0)
    for p in range(2):
        pltpu.make_async_copy(
            rows[p], out_hbm.at[pl.ds(rbase, _CHUNK_R)], wsem[p]).wait()


def _tc_body(t_ref, p1_ref, p2_ref, tdhi_ref, tdlo_ref, out_ref):
    ia = t_ref[...] * NB + p1_ref[...]
    ib = t_ref[...] * NB + p2_ref[...]
    lanes = lax.broadcasted_iota(jnp.int32, (_G, SEQ, 2 * HTAB), 2)
    target = jnp.where(lanes < HTAB, ia[:, :, None], ib[:, :, None] + HTAB)
    twohot = (lanes == target).astype(jnp.bfloat16).reshape(_G * SEQ, 2 * HTAB)
    res = (jnp.dot(twohot, tdhi_ref[...], preferred_element_type=jnp.float32)
           + jnp.dot(twohot, tdlo_ref[...], preferred_element_type=jnp.float32))
    out_ref[...] = res.reshape(_G, SEQ, D)


def kernel(x, token_embedding, pe):
    ctab, tdhi, tdlo = pl.pallas_call(
        _tables_body,
        out_shape=(jax.ShapeDtypeStruct((TAB, D), jnp.float32),
                   jax.ShapeDtypeStruct((2 * HTAB, D), jnp.bfloat16),
                   jax.ShapeDtypeStruct((2 * HTAB, D), jnp.bfloat16)),
    )(token_embedding[:NB], pe)

    sc_out = _sc_gather(x[:_KSC].reshape(-1), ctab)

    ntc = (BATCH - _KSC) // _G
    out = pl.pallas_call(
        _tc_body,
        grid=(ntc,),
        in_specs=[
            pl.BlockSpec((_G, SEQ), lambda i: (i, 0)),
            pl.BlockSpec((_G, SEQ), lambda i: (i, 0)),
            pl.BlockSpec((_G, SEQ), lambda i: (i, 0)),
            pl.BlockSpec((2 * HTAB, D), lambda i: (0, 0)),
            pl.BlockSpec((2 * HTAB, D), lambda i: (0, 0)),
        ],
        out_specs=pl.BlockSpec((_G, SEQ, D), lambda i: (_KSC // _G + i, 0, 0)),
        out_shape=jax.ShapeDtypeStruct((BATCH, SEQ, D), jnp.float32),
    )(x[_KSC:, 0, :], x[_KSC:, 1, :], x[_KSC:, 2, :], tdhi, tdlo)
    return lax.dynamic_update_slice(
        out, sc_out.reshape(_KSC, SEQ, D), (0, 0, 0))


# speedup vs baseline: 1.3412x; 1.0634x over previous
import functools

import jax
import jax.numpy as jnp
from jax import lax
from jax.experimental import pallas as pl
from jax.experimental.pallas import tpu as pltpu
from jax.experimental.pallas import tpu_sc as plsc

D = 256
HALF = 128
NB = 8
BATCH = 4096
SEQ = 64
NROWS = BATCH * SEQ
TAB = NB * NB * NB
HTAB = NB * NB

_KSC = 256
_G = 32

_info = plsc.get_sparse_core_info()
_NC, _NS = _info.num_cores, _info.num_subcores
_NW = _NC * _NS
_BPW = _KSC // _NW
_CHUNK_B = 2
_CHUNK_R = _CHUNK_B * SEQ
_NCHUNK = _BPW // _CHUNK_B
_XPW = _BPW * 3 * SEQ


def _tables_body(te_ref, pe_ref, c_ref, tdhi_ref, tdlo_ref):
    te8 = te_ref[...]
    pe = pe_ref[...]
    per = jnp.broadcast_to(pe[None, :, :], (NB, NB, HALF)).reshape(HTAB, HALF)
    ta = jnp.broadcast_to(te8[:, None, :HALF] * 16.0, (NB, NB, HALF)).reshape(HTAB, HALF)
    tb = jnp.broadcast_to(te8[:, None, HALF:] * 16.0, (NB, NB, HALF)).reshape(HTAB, HALF)
    a = ta + per
    b = tb + per
    ca = jnp.broadcast_to(a[:, None, :], (HTAB, NB, HALF)).reshape(TAB, HALF)
    cb = jnp.broadcast_to(b.reshape(NB, NB, HALF)[:, None, :, :],
                          (NB, NB, NB, HALF)).reshape(TAB, HALF)
    c_ref[...] = jnp.concatenate([ca, cb], axis=-1)
    z = jnp.zeros((HTAB, HALF), jnp.float32)
    td = jnp.concatenate(
        [jnp.concatenate([a, z], axis=1), jnp.concatenate([z, b], axis=1)],
        axis=0)
    hi = td.astype(jnp.bfloat16)
    tdhi_ref[...] = hi
    tdlo_ref[...] = (td - hi.astype(jnp.float32)).astype(jnp.bfloat16)


_mesh = plsc.VectorSubcoreMesh(core_axis_name="c", subcore_axis_name="s")


@functools.partial(
    pl.kernel,
    mesh=_mesh,
    out_type=jax.ShapeDtypeStruct((_KSC * SEQ, D), jnp.float32),
    scratch_types=[
        pltpu.VMEM((_XPW,), jnp.int32),
        pltpu.VMEM((2, _CHUNK_R), jnp.int32),
        pltpu.VMEM((_CHUNK_R, D), jnp.float32),
        pltpu.VMEM((_CHUNK_R, D), jnp.float32),
        pltpu.SemaphoreType.DMA,
        pltpu.SemaphoreType.DMA,
        pltpu.SemaphoreType.DMA,
    ],
)
def _sc_gather(xf_hbm, tab_hbm, out_hbm, xv, idxv, rowsA, rowsB,
               gsem, wsA, wsB):
    wid = lax.axis_index("s") * _NC + lax.axis_index("c")
    pltpu.sync_copy(xf_hbm.at[pl.ds(wid * _XPW, _XPW)], xv)
    rbase = wid * (_BPW * SEQ)

    rows = (rowsA, rowsB)
    wsem = (wsA, wsB)

    def do_chunk(c, p):
        @pl.when(c >= 2)
        def _():
            pltpu.make_async_copy(
                rows[p], out_hbm.at[pl.ds(rbase, _CHUNK_R)], wsem[p]).wait()

        for j in range(_CHUNK_B):
            boff = (c * _CHUNK_B + j) * (3 * SEQ)
            for k in range(SEQ // 16):
                t = xv[pl.ds(boff + k * 16, 16)]
                p1 = xv[pl.ds(boff + SEQ + k * 16, 16)]
                p2 = xv[pl.ds(boff + 2 * SEQ + k * 16, 16)]
                idxv[p, pl.ds(j * SEQ + k * 16, 16)] = t * (NB * NB) + p1 * NB + p2
        pltpu.async_copy(tab_hbm.at[idxv.at[p]], rows[p], gsem).wait()
        pltpu.async_copy(
            rows[p],
            out_hbm.at[pl.ds(rbase + c * _CHUNK_R, _CHUNK_R)],
            wsem[p])

    def body(i, carry):
        do_chunk(2 * i, 0)
        do_chunk(2 * i + 1, 1)
        return carry

    lax.fori_loop(0, _NCHUNK // 2, body, 0)
    for p in range(2):
        pltpu.make_async_copy(
            rows[p], out_hbm.at[pl.ds(rbase, _CHUNK_R)], wsem[p]).wait()


def _tc_body(t_ref, p1_ref, p2_ref, tdhi_ref, tdlo_ref, out_ref):
    ia = t_ref[...] * NB + p1_ref[...]
    ib = t_ref[...] * NB + p2_ref[...]
    lanes = lax.broadcasted_iota(jnp.int32, (_G, SEQ, 2 * HTAB), 2)
    target = jnp.where(lanes < HTAB, ia[:, :, None], ib[:, :, None] + HTAB)
    twohot = (lanes == target).astype(jnp.bfloat16).reshape(_G * SEQ, 2 * HTAB)
    res = (jnp.dot(twohot, tdhi_ref[...], preferred_element_type=jnp.float32)
           + jnp.dot(twohot, tdlo_ref[...], preferred_element_type=jnp.float32))
    out_ref[...] = res.reshape(_G, SEQ, D)


def kernel(x, token_embedding, pe):
    ctab, tdhi, tdlo = pl.pallas_call(
        _tables_body,
        out_shape=(jax.ShapeDtypeStruct((TAB, D), jnp.float32),
                   jax.ShapeDtypeStruct((2 * HTAB, D), jnp.bfloat16),
                   jax.ShapeDtypeStruct((2 * HTAB, D), jnp.bfloat16)),
    )(token_embedding[:NB], pe)

    sc_out = _sc_gather(x[:_KSC].reshape(-1), ctab)

    ntc = (BATCH - _KSC) // _G
    out = pl.pallas_call(
        _tc_body,
        grid=(ntc,),
        in_specs=[
            pl.BlockSpec((_G, SEQ), lambda i: (i, 0)),
            pl.BlockSpec((_G, SEQ), lambda i: (i, 0)),
            pl.BlockSpec((_G, SEQ), lambda i: (i, 0)),
            pl.BlockSpec((2 * HTAB, D), lambda i: (0, 0)),
            pl.BlockSpec((2 * HTAB, D), lambda i: (0, 0)),
        ],
        out_specs=pl.BlockSpec((_G, SEQ, D), lambda i: (_KSC // _G + i, 0, 0)),
        out_shape=jax.ShapeDtypeStruct((BATCH, SEQ, D), jnp.float32),
    )(x[_KSC:, 0, :], x[_KSC:, 1, :], x[_KSC:, 2, :], tdhi, tdlo)
    return lax.dynamic_update_slice(
        out, sc_out.reshape(_KSC, SEQ, D), (0, 0, 0))
